# Initial kernel scaffold; baseline (speedup 1.0000x reference)
import jax
import jax.numpy as jnp
from jax.experimental import pallas as pl


def kernel(view_indices, L):
    raise NotImplementedError("write your pallas kernel here")


# speedup vs baseline: 1.9841x; 1.9841x over previous
import functools

import jax
import jax.numpy as jnp
from jax import lax
from jax.experimental import pallas as pl
from jax.experimental.pallas import tpu as pltpu
from jax.experimental.pallas import tpu_sc as plsc

Q = 1000
QP = 1024
N = 4096
NC = 2
NS = 16
NW = NC * NS


@functools.lru_cache(maxsize=None)
def _make_sc_gather(V, D, B, CH):
    b_per_w = B // NW
    n_chunks = b_per_w // CH
    assert b_per_w % CH == 0 and CH % 8 == 0 and D % 16 == 0

    mesh = plsc.VectorSubcoreMesh(
        core_axis_name="c", subcore_axis_name="s",
        num_cores=NC, num_subcores=NS)

    @functools.partial(
        pl.kernel,
        out_type=jax.ShapeDtypeStruct((B, D), jnp.float32),
        mesh=mesh,
        scratch_types=[
            pltpu.VMEM((n_chunks, CH), jnp.int32),
            pltpu.VMEM((CH, D), jnp.float32),
            pltpu.VMEM((CH, D), jnp.float32),
            pltpu.SemaphoreType.DMA,
            pltpu.SemaphoreType.DMA,
        ],
    )
    def gather(table_hbm, idx_hbm, out_hbm, idx_v, buf0, buf1, sem0, sem1):
        wid = lax.axis_index("s") * NC + lax.axis_index("c")
        base = wid * b_per_w
        pltpu.sync_copy(idx_hbm.at[wid], idx_v)
        bufs = (buf0, buf1)
        sems = (sem0, sem1)
        handles = [None] * n_chunks
        handles[0] = pltpu.async_copy(
            table_hbm.at[idx_v.at[0]], bufs[0], sems[0])
        for c in range(n_chunks):
            handles[c].wait()
            if c + 1 < n_chunks:
                handles[c + 1] = pltpu.async_copy(
                    table_hbm.at[idx_v.at[c + 1]],
                    bufs[(c + 1) % 2], sems[(c + 1) % 2])
            pltpu.sync_copy(bufs[c % 2],
                            out_hbm.at[pl.ds(base + c * CH, CH)])

    return gather


def _matmul_body(l_ref, g_ref, out_ref):
    out_ref[...] = lax.dot_general(
        l_ref[...], g_ref[...], (((1,), (1,)), ((), ())),
        preferred_element_type=jnp.float32)


_matmul = pl.pallas_call(
    _matmul_body,
    grid=(4,),
    in_specs=[
        pl.BlockSpec((Q, QP), lambda j: (0, 0)),
        pl.BlockSpec((N // 4, QP), lambda j: (j, 0)),
    ],
    out_specs=pl.BlockSpec((Q, N // 4), lambda j: (0, j)),
    out_shape=jax.ShapeDtypeStruct((Q, N), jnp.float32),
)


def kernel(view_indices, L):
    vi = view_indices.astype(jnp.int32)
    idx_g = vi.reshape(NW, N // NW // 16, 16)
    idx_k = vi.reshape(NW, N // NW // 8, 8)
    Lp = jnp.pad(L, ((0, 0), (0, QP - Q)))
    G = _make_sc_gather(Q, QP, N, 16)(Lp, idx_g)
    C = _matmul(Lp, G)
    return _make_sc_gather(Q, N, N, 8)(C, idx_k)
